# baseline (speedup 1.0000x reference)
import functools

import jax
import jax.numpy as jnp
from jax import lax
from jax.experimental import pallas as pl
from jax.experimental.pallas import tpu as pltpu
from jax.experimental.pallas import tpu_sc as plsc

_EMBED = 32
_NC = 2
_NS = 16
_NW = _NC * _NS
_CH = 128
_NCH = 50
_BPW = _CH * _NCH
_B = _NW * _BPW


@functools.partial(
    pl.kernel,
    mesh=plsc.VectorSubcoreMesh(core_axis_name="c", subcore_axis_name="s"),
    out_type=jax.ShapeDtypeStruct((_B, _EMBED), jnp.float32),
    scratch_types=[
        pltpu.VMEM((_NCH, _CH), jnp.int32),
        pltpu.VMEM((_CH, _EMBED), jnp.float32),
        pltpu.SemaphoreType.DMA,
    ],
    compiler_params=pltpu.CompilerParams(use_tc_tiling_on_sc=False),
)
def _emb_gather(table_hbm, idx_hbm, out_hbm, idx_v, buf_v, gsem):
    wid = lax.axis_index("s") * _NC + lax.axis_index("c")
    pltpu.sync_copy(idx_hbm.at[wid], idx_v)
    base = wid * _BPW

    def body(j, carry):
        pltpu.async_copy(table_hbm.at[idx_v.at[j]], buf_v, gsem).wait()
        pltpu.sync_copy(buf_v, out_hbm.at[pl.ds(base + j * _CH, _CH)])
        return carry

    lax.fori_loop(0, _NCH, body, 0)


def kernel(inputs, embeddings):
    shape = inputs.shape
    idx3 = inputs.reshape(_NW, _NCH, _CH).astype(jnp.int32)
    out = _emb_gather(embeddings, idx3)
    return out.reshape(shape + (_EMBED,))


# speedup vs baseline: 1.2801x; 1.2801x over previous
import functools

import jax
import jax.numpy as jnp
from jax import lax
from jax.experimental import pallas as pl
from jax.experimental.pallas import tpu as pltpu
from jax.experimental.pallas import tpu_sc as plsc

_EMBED = 32
_NC = 2
_NS = 16
_NW = _NC * _NS
_CH = 128
_K = 10
_NG = 5
_NCH = _K * _NG
_BPW = _CH * _NCH
_B = _NW * _BPW


@functools.partial(
    pl.kernel,
    mesh=plsc.VectorSubcoreMesh(core_axis_name="c", subcore_axis_name="s"),
    out_type=jax.ShapeDtypeStruct((_NW * _NG, _K, _CH, _EMBED), jnp.float32),
    scratch_types=[
        pltpu.VMEM((_NCH, _CH), jnp.int32),
        pltpu.VMEM((_K, _CH, _EMBED), jnp.float32),
        pltpu.VMEM((_K, _CH, _EMBED), jnp.float32),
        pltpu.SemaphoreType.DMA,
        pltpu.SemaphoreType.DMA,
    ],
    compiler_params=pltpu.CompilerParams(use_tc_tiling_on_sc=False),
)
def _emb_gather(table_hbm, idx_hbm, out_hbm, idx_v, buf0, buf1, gsem, wsem):
    wid = lax.axis_index("s") * _NC + lax.axis_index("c")
    pltpu.sync_copy(idx_hbm.at[wid], idx_v)

    bufs = (buf0, buf1)
    wdesc = [None] * _NG
    for g in range(_NG):
        cur = bufs[g % 2]
        if g >= 2:
            wdesc[g - 2].wait()
        gd = [
            pltpu.async_copy(
                table_hbm.at[idx_v.at[g * _K + j]], cur.at[j], gsem
            )
            for j in range(_K)
        ]
        for d in gd:
            d.wait()
        wdesc[g] = pltpu.async_copy(cur, out_hbm.at[wid * _NG + g], wsem)
    wdesc[_NG - 2].wait()
    wdesc[_NG - 1].wait()


def kernel(inputs, embeddings):
    shape = inputs.shape
    idx3 = inputs.reshape(_NW, _NCH, _CH).astype(jnp.int32)
    out = _emb_gather(embeddings, idx3)
    return out.reshape(shape + (_EMBED,))


# speedup vs baseline: 1.2825x; 1.0018x over previous
import functools

import jax
import jax.numpy as jnp
from jax import lax
from jax.experimental import pallas as pl
from jax.experimental.pallas import tpu as pltpu
from jax.experimental.pallas import tpu_sc as plsc

_EMBED = 32
_NC = 2
_NS = 16
_NW = _NC * _NS
_CH = 128
_K = 5
_NG = 10
_NCH = _K * _NG
_BPW = _CH * _NCH
_B = _NW * _BPW


@functools.partial(
    pl.kernel,
    mesh=plsc.VectorSubcoreMesh(core_axis_name="c", subcore_axis_name="s"),
    out_type=jax.ShapeDtypeStruct((_NW * _NG, _K, _CH, _EMBED), jnp.float32),
    scratch_types=[
        pltpu.VMEM((_NCH, _CH), jnp.int32),
        pltpu.VMEM((_K, _CH, _EMBED), jnp.float32),
        pltpu.VMEM((_K, _CH, _EMBED), jnp.float32),
        pltpu.VMEM((_K, _CH, _EMBED), jnp.float32),
        pltpu.SemaphoreType.DMA,
        pltpu.SemaphoreType.DMA,
        pltpu.SemaphoreType.DMA,
        pltpu.SemaphoreType.DMA,
    ],
    compiler_params=pltpu.CompilerParams(use_tc_tiling_on_sc=False),
)
def _emb_gather(
    table_hbm, idx_hbm, out_hbm, idx_v, buf0, buf1, buf2, gsem0, gsem1, gsem2, wsem
):
    wid = lax.axis_index("s") * _NC + lax.axis_index("c")
    pltpu.sync_copy(idx_hbm.at[wid], idx_v)

    bufs = (buf0, buf1, buf2)
    gsems = (gsem0, gsem1, gsem2)

    def issue_group(g, buf):
        return [
            pltpu.async_copy(
                table_hbm.at[idx_v.at[g * _K + j]], buf.at[j], gsems[g % 3]
            )
            for j in range(_K)
        ]

    gdesc = [None] * _NG
    wdesc = [None] * _NG
    gdesc[0] = issue_group(0, bufs[0])
    gdesc[1] = issue_group(1, bufs[1])
    for g in range(_NG):
        cur = bufs[g % 3]
        for d in gdesc[g]:
            d.wait()
        wdesc[g] = pltpu.async_copy(cur, out_hbm.at[wid * _NG + g], wsem)
        if g + 2 < _NG:
            if g >= 1:
                wdesc[g - 1].wait()
            gdesc[g + 2] = issue_group(g + 2, bufs[(g + 2) % 3])
    wdesc[_NG - 2].wait()
    wdesc[_NG - 1].wait()


def kernel(inputs, embeddings):
    shape = inputs.shape
    idx3 = inputs.reshape(_NW, _NCH, _CH).astype(jnp.int32)
    out = _emb_gather(embeddings, idx3)
    return out.reshape(shape + (_EMBED,))
